# baseline (speedup 1.0000x reference)
import functools

import jax
import jax.numpy as jnp
from jax import lax
from jax.experimental import pallas as pl
from jax.experimental.pallas import tpu as pltpu
from jax.experimental.pallas import tpu_sc as plsc

_N_TOKEN = 100000
_H = 64
_NG = 4
_B = 1024
_BN = 2048
_KB = 4
_BW = _KB * _BN
_NSTEP = 13
_LAST = _NSTEP - 1
_TAILW = 1664


def _sc_gather(emb, idx):
    info = plsc.get_sparse_core_info()
    nc, ns = info.num_cores, info.num_subcores
    nw = nc * ns
    btot = idx.shape[0]
    b_per_w = btot // nw
    mesh = plsc.VectorSubcoreMesh(core_axis_name="c", subcore_axis_name="s")

    @functools.partial(
        pl.kernel,
        mesh=mesh,
        out_type=jax.ShapeDtypeStruct((btot, _H), jnp.float32),
        scratch_types=[
            pltpu.VMEM((b_per_w,), jnp.int32),
            pltpu.VMEM((b_per_w, _H), jnp.float32),
            pltpu.SemaphoreType.DMA,
        ],
        compiler_params=pltpu.CompilerParams(use_tc_tiling_on_sc=False),
    )
    def gather_k(table_hbm, idx_hbm, out_hbm, idx_v, rows_v, sem):
        wid = lax.axis_index("s") * nc + lax.axis_index("c")
        base = wid * b_per_w
        pltpu.sync_copy(idx_hbm.at[pl.ds(base, b_per_w)], idx_v)
        pltpu.async_copy(table_hbm.at[idx_v], rows_v, sem).wait()
        pltpu.sync_copy(rows_v, out_hbm.at[pl.ds(base, b_per_w)])

    return gather_k(emb, idx)


def _decoder_body(flat_ref, w_ref, b_ref, emb_ref, out_ref, sliv_ref,
                  hid_ref, bufa, bufb, bufc, bufd, sems):
    i = pl.program_id(0)
    bufs = (bufa, bufb, bufc, bufd)

    @pl.when(i == 0)
    def _():
        hid = lax.dot_general(
            flat_ref[...], w_ref[...],
            (((1,), (1,)), ((), ())),
            preferred_element_type=jnp.float32,
        )
        hid_ref[...] = hid + b_ref[...]

    def _block(k):
        return lax.dot_general(
            hid_ref[...], emb_ref[pl.ds(k * _BN, _BN), :],
            (((1,), (1,)), ((), ())),
            preferred_element_type=jnp.float32,
        )

    @pl.when(i >= 1)
    def _():
        pltpu.make_async_copy(
            bufa, out_ref.at[:, pl.ds(0, _BN)], sems.at[0]).wait()
    bufa[...] = _block(0)

    @pl.when(i < _LAST)
    def _():
        pltpu.make_async_copy(
            bufa, out_ref.at[:, pl.ds(i * _BW, _BN)], sems.at[0]).start()

    for k in range(1, _KB):
        @pl.when(i < _LAST)
        def _(k=k):
            @pl.when(i >= 1)
            def _():
                pltpu.make_async_copy(
                    bufs[k], out_ref.at[:, pl.ds(0, _BN)], sems.at[k]).wait()
            bufs[k][...] = _block(k)
            pltpu.make_async_copy(
                bufs[k], out_ref.at[:, pl.ds(i * _BW + k * _BN, _BN)],
                sems.at[k]).start()

    @pl.when(i == _LAST)
    def _():
        sliv_ref[...] = bufa[:, _TAILW:_TAILW + 32]
        pltpu.make_async_copy(
            bufa.at[:, pl.ds(0, _TAILW)],
            out_ref.at[:, pl.ds(_LAST * _BW, _TAILW)],
            sems.at[0],
        ).start()
        for k in range(1, _KB):
            pltpu.make_async_copy(
                bufs[k], out_ref.at[:, pl.ds(0, _BN)], sems.at[k]).wait()
        pltpu.make_async_copy(
            bufa.at[:, pl.ds(0, _TAILW)],
            out_ref.at[:, pl.ds(0, _TAILW)],
            sems.at[0],
        ).wait()


def kernel(x, emb, fc1_w, fc1_b):
    idx = x.reshape(-1).astype(jnp.int32)
    gathered = _sc_gather(emb, idx)
    flat = gathered.reshape(_B, _NG * _H)

    out, sliver = pl.pallas_call(
        _decoder_body,
        grid=(_NSTEP,),
        in_specs=[
            pl.BlockSpec((_B, _NG * _H), lambda i: (0, 0)),
            pl.BlockSpec((_H, _NG * _H), lambda i: (0, 0)),
            pl.BlockSpec((1, _H), lambda i: (0, 0)),
            pl.BlockSpec((_BW, _H), lambda i: (i, 0)),
        ],
        out_specs=[pl.BlockSpec(memory_space=pl.ANY),
                   pl.BlockSpec((_B, 32), lambda i: (0, 0))],
        out_shape=[jax.ShapeDtypeStruct((_B, _N_TOKEN), jnp.float32),
                   jax.ShapeDtypeStruct((_B, 32), jnp.float32)],
        scratch_shapes=[
            pltpu.VMEM((_B, _H), jnp.float32),
            pltpu.VMEM((_B, _BN), jnp.float32),
            pltpu.VMEM((_B, _BN), jnp.float32),
            pltpu.VMEM((_B, _BN), jnp.float32),
            pltpu.VMEM((_B, _BN), jnp.float32),
            pltpu.SemaphoreType.DMA((_KB,)),
        ],
        compiler_params=pltpu.CompilerParams(
            dimension_semantics=("arbitrary",),
            vmem_limit_bytes=60 * 1024 * 1024,
        ),
    )(flat, fc1_w, fc1_b.reshape(1, _H), emb)
    return lax.dynamic_update_slice(out, sliver, (0, _LAST * _BW + _TAILW))
